# baseline (speedup 1.0000x reference)
import jax
import jax.numpy as jnp
from jax.experimental import pallas as pl

_SLOPE = (1.0 / 8.0 + 1.0 / 3.0) / 2.0


def _body(x_ref, wp_ref, wn_ref, b_ref, o_ref):
    x = x_ref[...]
    p = jnp.maximum(x, 0.0).astype(jnp.bfloat16)
    n = jnp.minimum(x, 0.0).astype(jnp.bfloat16)
    acc = jnp.dot(p, wp_ref[...], preferred_element_type=jnp.float32)
    acc += jnp.dot(n, wn_ref[...], preferred_element_type=jnp.float32)
    o_ref[...] = acc + b_ref[...]


def kernel(node_feats, edge_feats, edge_index, W1, b1, lW1, lb1, W2, b2, lW2, lb2, oW, ob):
    n, d = node_feats.shape
    h = oW.shape[1]
    block = 1000
    grid = (n // block,)
    w1 = oW[:d]
    w2 = oW[d:]
    wp = (w1 + w2).astype(jnp.bfloat16)
    wn = (_SLOPE * w1 + (_SLOPE * _SLOPE) * w2).astype(jnp.bfloat16)
    b = ob.reshape(1, h)
    return pl.pallas_call(
        _body,
        grid=grid,
        in_specs=[
            pl.BlockSpec((block, d), lambda i: (i, 0)),
            pl.BlockSpec((d, h), lambda i: (0, 0)),
            pl.BlockSpec((d, h), lambda i: (0, 0)),
            pl.BlockSpec((1, h), lambda i: (0, 0)),
        ],
        out_specs=pl.BlockSpec((block, h), lambda i: (i, 0)),
        out_shape=jax.ShapeDtypeStruct((n, h), jnp.float32),
    )(node_feats, wp, wn, b)


# speedup vs baseline: 1.2493x; 1.2493x over previous
import jax
import jax.numpy as jnp
from jax.experimental import pallas as pl

_SLOPE = (1.0 / 8.0 + 1.0 / 3.0) / 2.0


def _body(x_ref, wp_ref, wn_ref, b_ref, o_ref):
    x = x_ref[...]
    p = jnp.maximum(x, 0.0).astype(jnp.bfloat16)
    n = jnp.minimum(x, 0.0).astype(jnp.bfloat16)
    acc = jnp.dot(p, wp_ref[...], preferred_element_type=jnp.float32)
    acc += jnp.dot(n, wn_ref[...], preferred_element_type=jnp.float32)
    o_ref[...] = acc + b_ref[...]


def kernel(node_feats, edge_feats, edge_index, W1, b1, lW1, lb1, W2, b2, lW2, lb2, oW, ob):
    n, d = node_feats.shape
    h = oW.shape[1]
    block = 2000
    grid = (n // block,)
    w1 = oW[:d]
    w2 = oW[d:]
    wp = (w1 + w2).astype(jnp.bfloat16)
    wn = (_SLOPE * w1 + (_SLOPE * _SLOPE) * w2).astype(jnp.bfloat16)
    b = ob.reshape(1, h)
    return pl.pallas_call(
        _body,
        grid=grid,
        in_specs=[
            pl.BlockSpec((block, d), lambda i: (i, 0)),
            pl.BlockSpec((d, h), lambda i: (0, 0)),
            pl.BlockSpec((d, h), lambda i: (0, 0)),
            pl.BlockSpec((1, h), lambda i: (0, 0)),
        ],
        out_specs=pl.BlockSpec((block, h), lambda i: (i, 0)),
        out_shape=jax.ShapeDtypeStruct((n, h), jnp.float32),
    )(node_feats, wp, wn, b)
